# baseline (speedup 1.0000x reference)
import jax
import jax.numpy as jnp
from jax.experimental import pallas as pl
from jax.experimental.pallas import tpu as pltpu

_NCH = 16


def _copy_body(tab_ref, out_ref, buf, in_sems, out_sems):
    s = out_ref.shape[0]
    ch = s // _NCH
    ins, outs = [], []
    for i in range(_NCH):
        c = pltpu.make_async_copy(
            tab_ref.at[pl.ds(i * ch, ch)], buf.at[i, :, 0], in_sems.at[i])
        c.start()
        ins.append(c)
    for i in range(_NCH):
        ins[i].wait()
        c = pltpu.make_async_copy(
            buf.at[i], out_ref.at[pl.ds(i * ch, ch)], out_sems.at[i])
        c.start()
        outs.append(c)
    for c in outs:
        c.wait()


def kernel(x, pos_table):
    s = x.shape[0]
    n, e = pos_table.shape
    out = pl.pallas_call(
        _copy_body,
        in_specs=[pl.BlockSpec(memory_space=pl.ANY)],
        out_specs=pl.BlockSpec(memory_space=pl.ANY),
        out_shape=jax.ShapeDtypeStruct((s, 1, e), pos_table.dtype),
        scratch_shapes=[
            pltpu.VMEM((_NCH, s // _NCH, 1, e), pos_table.dtype),
            pltpu.SemaphoreType.DMA((_NCH,)),
            pltpu.SemaphoreType.DMA((_NCH,)),
        ],
    )(pos_table)
    return out


# speedup vs baseline: 1.0247x; 1.0247x over previous
import jax
import jax.numpy as jnp
from jax.experimental import pallas as pl
from jax.experimental.pallas import tpu as pltpu

_NCH = 4


def _copy_body(tab_ref, out_ref, buf, in_sems, out_sems):
    s = out_ref.shape[0]
    ch = s // _NCH
    ins, outs = [], []
    for i in range(_NCH):
        c = pltpu.make_async_copy(
            tab_ref.at[pl.ds(i * ch, ch)], buf.at[i, :, 0], in_sems.at[i])
        c.start()
        ins.append(c)
    for i in range(_NCH):
        ins[i].wait()
        c = pltpu.make_async_copy(
            buf.at[i], out_ref.at[pl.ds(i * ch, ch)], out_sems.at[i])
        c.start()
        outs.append(c)
    for c in outs:
        c.wait()


def kernel(x, pos_table):
    s = x.shape[0]
    n, e = pos_table.shape
    out = pl.pallas_call(
        _copy_body,
        in_specs=[pl.BlockSpec(memory_space=pl.ANY)],
        out_specs=pl.BlockSpec(memory_space=pl.ANY),
        out_shape=jax.ShapeDtypeStruct((s, 1, e), pos_table.dtype),
        scratch_shapes=[
            pltpu.VMEM((_NCH, s // _NCH, 1, e), pos_table.dtype),
            pltpu.SemaphoreType.DMA((_NCH,)),
            pltpu.SemaphoreType.DMA((_NCH,)),
        ],
    )(pos_table)
    return out
